# Initial kernel scaffold; baseline (speedup 1.0000x reference)
import jax
import jax.numpy as jnp
from jax.experimental import pallas as pl


def kernel(x, edge_index, edge_weights, W1, b1, W2, b2, bn_gamma, bn_beta, fc1_W, fc1_b, fc2_W, fc2_b):
    raise NotImplementedError("write your pallas kernel here")


# speedup vs baseline: 7.6206x; 7.6206x over previous
import functools
import math

import jax
import jax.numpy as jnp
from jax import lax
from jax.experimental import pallas as pl
from jax.experimental.pallas import tpu as pltpu
from jax.experimental.pallas import tpu_sc as plsc

N = 10000
E = 320000
D = 128

NC = 2
NS = 16
NW = NC * NS
EPW = E // NW
WIN = 80
NWIN = EPW // WIN
NP = 10240
RPT = NP // NS

ALPHA = 0.5
BETA1 = math.log(1.0 / 1.0 + 1.0)
BETA2 = math.log(1.0 / 2.0 + 1.0)

_mesh = plsc.VectorSubcoreMesh(core_axis_name="c", subcore_axis_name="s")


@functools.partial(
    pl.kernel,
    out_type=jax.ShapeDtypeStruct((NC, NP), jnp.float32),
    mesh=_mesh,
    scratch_types=[
        pltpu.VMEM((NWIN, WIN), jnp.int32),
        pltpu.VMEM((WIN,), jnp.float32),
        pltpu.VMEM_SHARED((NP,), jnp.float32),
    ],
)
def _deg_sc(dst_hbm, zeros_hbm, out_hbm, idx_v, ones_v, acc_sh):
    c = lax.axis_index("c")
    s = lax.axis_index("s")
    w = c * NS + s
    pltpu.sync_copy(zeros_hbm.at[pl.ds(s * RPT, RPT)], acc_sh.at[pl.ds(s * RPT, RPT)])
    for i in range(WIN // 16):
        ones_v[pl.ds(i * 16, 16)] = jnp.ones((16,), jnp.float32)
    pltpu.sync_copy(dst_hbm.at[w], idx_v)
    plsc.subcore_barrier()

    def body(j, carry):
        pltpu.sync_copy(ones_v, acc_sh.at[idx_v.at[j]], add=True)
        return carry

    lax.fori_loop(0, NWIN, body, 0)
    plsc.subcore_barrier()
    pltpu.sync_copy(acc_sh.at[pl.ds(s * RPT, RPT)], out_hbm.at[c, pl.ds(s * RPT, RPT)])


@functools.partial(
    pl.kernel,
    out_type=jax.ShapeDtypeStruct((NC, NP, D), jnp.float32),
    mesh=_mesh,
    scratch_types=[
        pltpu.VMEM((NWIN, WIN), jnp.int32),
        pltpu.VMEM((NWIN, WIN), jnp.int32),
        pltpu.VMEM((WIN, D), jnp.float32),
        pltpu.VMEM_SHARED((NP, D), jnp.float32),
        pltpu.SemaphoreType.DMA,
    ],
)
def _agg_sc(h_hbm, src_hbm, dst_hbm, zeros_hbm, out_hbm,
            src_v, dst_v, rows_v, acc_sh, gsem):
    c = lax.axis_index("c")
    s = lax.axis_index("s")
    w = c * NS + s
    pltpu.sync_copy(zeros_hbm.at[pl.ds(s * RPT, RPT)], acc_sh.at[pl.ds(s * RPT, RPT)])
    pltpu.sync_copy(src_hbm.at[w], src_v)
    pltpu.sync_copy(dst_hbm.at[w], dst_v)
    plsc.subcore_barrier()

    def body(j, carry):
        pltpu.async_copy(h_hbm.at[src_v.at[j]], rows_v, gsem).wait()
        pltpu.sync_copy(rows_v, acc_sh.at[dst_v.at[j]], add=True)
        return carry

    lax.fori_loop(0, NWIN, body, 0)
    plsc.subcore_barrier()
    pltpu.sync_copy(acc_sh.at[pl.ds(s * RPT, RPT)], out_hbm.at[c, pl.ds(s * RPT, RPT)])


def _leaky(v):
    return jnp.where(v >= 0, v, 0.01 * v)


def _norm_from_deg(deg_ref):
    deg = deg_ref[0, :N] + deg_ref[1, :N]
    return jnp.where(deg > 0, lax.rsqrt(jnp.maximum(deg, 1.0)), 0.0)


def _tc1_body(deg_ref, x_ref, h1n_ref):
    norm = _norm_from_deg(deg_ref)
    h1n_ref[...] = x_ref[...] * norm[:, None]


_tc1 = pl.pallas_call(
    _tc1_body,
    out_shape=jax.ShapeDtypeStruct((N, D), jnp.float32),
)


def _tc2_body(aggp_ref, x_ref, deg_ref, W1_ref, b1_ref, o1_ref, h2n_ref):
    norm = _norm_from_deg(deg_ref)
    agg = (aggp_ref[0, :N] + aggp_ref[1, :N]) * norm[:, None]
    t = (1.0 - ALPHA) * agg + ALPHA * x_ref[...]
    z = (1.0 - BETA1) * t + BETA1 * jnp.dot(
        t, W1_ref[...], preferred_element_type=jnp.float32) + b1_ref[...][None, :]
    o1 = _leaky(z)
    o1_ref[...] = o1
    h2n_ref[...] = o1 * norm[:, None]


_tc2 = pl.pallas_call(
    _tc2_body,
    out_shape=[
        jax.ShapeDtypeStruct((N, D), jnp.float32),
        jax.ShapeDtypeStruct((N, D), jnp.float32),
    ],
)


def _tc3_body(aggp_ref, o1_ref, deg_ref, W2_ref, b2_ref, g_ref, bb_ref,
              f1w_ref, f1b_ref, f2w_ref, f2b_ref, out_ref):
    norm = _norm_from_deg(deg_ref)
    agg = (aggp_ref[0, :N] + aggp_ref[1, :N]) * norm[:, None]
    t = (1.0 - ALPHA) * agg + ALPHA * o1_ref[...]
    h = (1.0 - BETA2) * t + BETA2 * jnp.dot(
        t, W2_ref[...], preferred_element_type=jnp.float32) + b2_ref[...][None, :]
    mean = jnp.mean(h, axis=0)
    var = jnp.mean((h - mean[None, :]) ** 2, axis=0)
    hb = (h - mean[None, :]) / jnp.sqrt(var + 1e-5)[None, :] * g_ref[...][None, :] \
        + bb_ref[...][None, :]
    hb = _leaky(hb)
    pooled = jnp.sum(hb, axis=0, keepdims=True)
    u = _leaky(jnp.dot(pooled, f1w_ref[...], preferred_element_type=jnp.float32)
               + f1b_ref[...][None, :])
    out_ref[...] = jnp.dot(u, f2w_ref[...], preferred_element_type=jnp.float32) \
        + f2b_ref[...][None, :]


_tc3 = pl.pallas_call(
    _tc3_body,
    out_shape=jax.ShapeDtypeStruct((1, 2), jnp.float32),
)


def kernel(x, edge_index, edge_weights, W1, b1, W2, b2, bn_gamma, bn_beta,
           fc1_W, fc1_b, fc2_W, fc2_b):
    del edge_weights
    src = jnp.reshape(edge_index[0], (NW, NWIN, WIN))
    dst = jnp.reshape(edge_index[1], (NW, NWIN, WIN))
    zeros_n = jnp.zeros((NP,), jnp.float32)
    zeros_nd = jnp.zeros((NP, D), jnp.float32)

    deg_parts = _deg_sc(dst, zeros_n)
    h1n = _tc1(deg_parts, x)
    agg1 = _agg_sc(h1n, src, dst, zeros_nd)
    o1, h2n = _tc2(agg1, x, deg_parts, W1, b1)
    agg2 = _agg_sc(h2n, src, dst, zeros_nd)
    out = _tc3(agg2, o1, deg_parts, W2, b2, bn_gamma, bn_beta,
               fc1_W, fc1_b, fc2_W, fc2_b)
    return out


# speedup vs baseline: 11.8594x; 1.5562x over previous
import functools
import math

import jax
import jax.numpy as jnp
from jax import lax
from jax.experimental import pallas as pl
from jax.experimental.pallas import tpu as pltpu
from jax.experimental.pallas import tpu_sc as plsc

N = 10000
E = 320000
D = 128

NC = 2
NS = 16
NW = NC * NS
EPW = E // NW
WIN = 80
NWIN = EPW // WIN
NP = 10240
RPT = NP // NS

ALPHA = 0.5
BETA1 = math.log(1.0 / 1.0 + 1.0)
BETA2 = math.log(1.0 / 2.0 + 1.0)

_mesh = plsc.VectorSubcoreMesh(core_axis_name="c", subcore_axis_name="s")


@functools.partial(
    pl.kernel,
    out_type=jax.ShapeDtypeStruct((NC, NP), jnp.float32),
    mesh=_mesh,
    scratch_types=[
        pltpu.VMEM((NWIN, WIN), jnp.int32),
        pltpu.VMEM((WIN,), jnp.float32),
        pltpu.VMEM_SHARED((NP,), jnp.float32),
    ],
)
def _deg_sc(dst_hbm, zeros_hbm, out_hbm, idx_v, ones_v, acc_sh):
    c = lax.axis_index("c")
    s = lax.axis_index("s")
    w = c * NS + s
    pltpu.sync_copy(zeros_hbm.at[pl.ds(s * RPT, RPT)], acc_sh.at[pl.ds(s * RPT, RPT)])
    for i in range(WIN // 16):
        ones_v[pl.ds(i * 16, 16)] = jnp.ones((16,), jnp.float32)
    pltpu.sync_copy(dst_hbm.at[w], idx_v)
    plsc.subcore_barrier()

    def body(j, carry):
        pltpu.sync_copy(ones_v, acc_sh.at[idx_v.at[j]], add=True)
        return carry

    lax.fori_loop(0, NWIN, body, 0)
    plsc.subcore_barrier()
    pltpu.sync_copy(acc_sh.at[pl.ds(s * RPT, RPT)], out_hbm.at[c, pl.ds(s * RPT, RPT)])


@functools.partial(
    pl.kernel,
    out_type=jax.ShapeDtypeStruct((NC, NP, D), jnp.float32),
    mesh=_mesh,
    scratch_types=[
        pltpu.VMEM((NWIN, WIN), jnp.int32),
        pltpu.VMEM((NWIN, WIN), jnp.int32),
        pltpu.VMEM((2, WIN, D), jnp.float32),
        pltpu.VMEM_SHARED((NP, D), jnp.float32),
        pltpu.SemaphoreType.DMA,
        pltpu.SemaphoreType.DMA,
    ],
    compiler_params=pltpu.CompilerParams(use_tc_tiling_on_sc=False),
)
def _agg_sc(h_hbm, src_hbm, dst_hbm, zeros_hbm, out_hbm,
            src_v, dst_v, rows_v, acc_sh, gsem0, gsem1):
    c = lax.axis_index("c")
    s = lax.axis_index("s")
    w = c * NS + s
    pltpu.sync_copy(zeros_hbm.at[pl.ds(s * RPT, RPT)], acc_sh.at[pl.ds(s * RPT, RPT)])
    pltpu.sync_copy(src_hbm.at[w], src_v)
    pltpu.sync_copy(dst_hbm.at[w], dst_v)
    plsc.subcore_barrier()

    def _start(j, b, sem):
        pltpu.async_copy(h_hbm.at[src_v.at[j]], rows_v.at[b], sem)

    def _drain(j, b, sem):
        pltpu.make_async_copy(h_hbm.at[src_v.at[j]], rows_v.at[b], sem).wait()
        pltpu.sync_copy(rows_v.at[b], acc_sh.at[dst_v.at[j]], add=True)

    _start(0, 0, gsem0)
    _start(1, 1, gsem1)

    def body(i, carry):
        j = 2 * i
        _drain(j, 0, gsem0)
        _start(j + 2, 0, gsem0)
        _drain(j + 1, 1, gsem1)
        _start(j + 3, 1, gsem1)
        return carry

    lax.fori_loop(0, (NWIN - 3) // 2, body, 0)
    _drain(NWIN - 3, 0, gsem0)
    _start(NWIN - 1, 0, gsem0)
    _drain(NWIN - 2, 1, gsem1)
    _drain(NWIN - 1, 0, gsem0)
    plsc.subcore_barrier()
    pltpu.sync_copy(acc_sh.at[pl.ds(s * RPT, RPT)], out_hbm.at[c, pl.ds(s * RPT, RPT)])


def _leaky(v):
    return jnp.where(v >= 0, v, 0.01 * v)


def _norm_from_deg(deg_ref):
    deg = deg_ref[0, :N] + deg_ref[1, :N]
    return jnp.where(deg > 0, lax.rsqrt(jnp.maximum(deg, 1.0)), 0.0)


def _tc1_body(deg_ref, x_ref, h1n_ref):
    norm = _norm_from_deg(deg_ref)
    h1n_ref[...] = x_ref[...] * norm[:, None]


_tc1 = pl.pallas_call(
    _tc1_body,
    out_shape=jax.ShapeDtypeStruct((N, D), jnp.float32),
)


def _tc2_body(aggp_ref, x_ref, deg_ref, W1_ref, b1_ref, o1_ref, h2n_ref):
    norm = _norm_from_deg(deg_ref)
    agg = (aggp_ref[0, :N] + aggp_ref[1, :N]) * norm[:, None]
    t = (1.0 - ALPHA) * agg + ALPHA * x_ref[...]
    z = (1.0 - BETA1) * t + BETA1 * jnp.dot(
        t, W1_ref[...], preferred_element_type=jnp.float32) + b1_ref[...][None, :]
    o1 = _leaky(z)
    o1_ref[...] = o1
    h2n_ref[...] = o1 * norm[:, None]


_tc2 = pl.pallas_call(
    _tc2_body,
    out_shape=[
        jax.ShapeDtypeStruct((N, D), jnp.float32),
        jax.ShapeDtypeStruct((N, D), jnp.float32),
    ],
)


def _tc3_body(aggp_ref, o1_ref, deg_ref, W2_ref, b2_ref, g_ref, bb_ref,
              f1w_ref, f1b_ref, f2w_ref, f2b_ref, out_ref):
    norm = _norm_from_deg(deg_ref)
    agg = (aggp_ref[0, :N] + aggp_ref[1, :N]) * norm[:, None]
    t = (1.0 - ALPHA) * agg + ALPHA * o1_ref[...]
    h = (1.0 - BETA2) * t + BETA2 * jnp.dot(
        t, W2_ref[...], preferred_element_type=jnp.float32) + b2_ref[...][None, :]
    mean = jnp.mean(h, axis=0)
    var = jnp.mean((h - mean[None, :]) ** 2, axis=0)
    hb = (h - mean[None, :]) / jnp.sqrt(var + 1e-5)[None, :] * g_ref[...][None, :] \
        + bb_ref[...][None, :]
    hb = _leaky(hb)
    pooled = jnp.sum(hb, axis=0, keepdims=True)
    u = _leaky(jnp.dot(pooled, f1w_ref[...], preferred_element_type=jnp.float32)
               + f1b_ref[...][None, :])
    out_ref[...] = jnp.dot(u, f2w_ref[...], preferred_element_type=jnp.float32) \
        + f2b_ref[...][None, :]


_tc3 = pl.pallas_call(
    _tc3_body,
    out_shape=jax.ShapeDtypeStruct((1, 2), jnp.float32),
)


def kernel(x, edge_index, edge_weights, W1, b1, W2, b2, bn_gamma, bn_beta,
           fc1_W, fc1_b, fc2_W, fc2_b):
    del edge_weights
    src = jnp.reshape(edge_index[0], (NW, NWIN, WIN))
    dst = jnp.reshape(edge_index[1], (NW, NWIN, WIN))
    zeros_n = jnp.zeros((NP,), jnp.float32)
    zeros_nd = jnp.zeros((NP, D), jnp.float32)

    deg_parts = _deg_sc(dst, zeros_n)
    h1n = _tc1(deg_parts, x)
    agg1 = _agg_sc(h1n, src, dst, zeros_nd)
    o1, h2n = _tc2(agg1, x, deg_parts, W1, b1)
    agg2 = _agg_sc(h2n, src, dst, zeros_nd)
    out = _tc3(agg2, o1, deg_parts, W2, b2, bn_gamma, bn_beta,
               fc1_W, fc1_b, fc2_W, fc2_b)
    return out
